# baseline (speedup 1.0000x reference)
import functools

import jax
import jax.numpy as jnp
from jax import lax
from jax.experimental import pallas as pl
from jax.experimental.pallas import tpu as pltpu
from jax.experimental.pallas import tpu_sc as plsc

N_NODES = 10000
N_EDGES = 320000
D = 128
NC, NS, L = 2, 16, 16
NW = NC * NS
CHUNK = 64
N_CHUNKS = 160
E_PER_W = N_CHUNKS * CHUNK
E_PAD = NW * E_PER_W
N_ACC = N_NODES + 16
R8 = (N_NODES // NS) // 8 * 8
TAIL = N_NODES - R8 * NS
ZTAIL = N_ACC - R8 * NS


def _sc_aggregate(x, src, dst, zrows):
    mesh = plsc.VectorSubcoreMesh(core_axis_name="c", subcore_axis_name="s")

    @functools.partial(
        pl.kernel,
        out_type=[
            jax.ShapeDtypeStruct((NC, N_NODES, D), jnp.float32),
            jax.ShapeDtypeStruct((NW, 1, N_NODES), jnp.float32),
        ],
        mesh=mesh,
        compiler_params=pltpu.CompilerParams(needs_layout_passes=False),
        scratch_types=[
            [pltpu.VMEM((CHUNK,), jnp.int32)] * 2,
            [pltpu.VMEM((CHUNK,), jnp.int32)] * 2,
            [pltpu.VMEM((CHUNK, D), jnp.float32)] * 2,
            pltpu.VMEM((N_ACC,), jnp.float32),
            [pltpu.SemaphoreType.DMA] * 2,
            [pltpu.SemaphoreType.DMA] * 2,
            pltpu.VMEM_SHARED((N_ACC, D), jnp.float32),
        ],
    )
    def k(x_hbm, src_hbm, dst_hbm, z_hbm, part_hbm, degp_hbm,
          srcv, dstv, rows, deg_v, isem, gsem, agg_sh):
        cid = lax.axis_index("c")
        sid = lax.axis_index("s")
        wid = sid * NC + cid

        pltpu.sync_copy(z_hbm, agg_sh.at[pl.ds(sid * R8, R8)])

        @pl.when(sid == 0)
        def _():
            pltpu.sync_copy(z_hbm.at[pl.ds(0, ZTAIL)],
                            agg_sh.at[pl.ds(NS * R8, ZTAIL)])

        def zdeg(i, carry):
            deg_v[pl.ds(i * L, L)] = jnp.zeros((L,), jnp.float32)
            return carry

        lax.fori_loop(0, N_ACC // L, zdeg, 0)
        plsc.subcore_barrier()

        ones = jnp.ones((L,), jnp.float32)

        def deg_update(b):
            def deg_body(j, c2):
                idx = dstv[b][pl.ds(j * L, L)]
                plsc.addupdate_scatter(deg_v, [idx], ones)
                return c2

            lax.fori_loop(0, CHUNK // L, deg_body, 0)

        def idx_load(c, b):
            base = wid * E_PER_W + c * CHUNK
            pltpu.async_copy(src_hbm.at[pl.ds(base, CHUNK)], srcv[b], isem[b])
            pltpu.async_copy(dst_hbm.at[pl.ds(base, CHUNK)], dstv[b], isem[b])

        def idx_wait(b):
            pltpu.make_async_copy(src_hbm.at[pl.ds(0, CHUNK)], srcv[b], isem[b]).wait()
            pltpu.make_async_copy(dst_hbm.at[pl.ds(0, CHUNK)], dstv[b], isem[b]).wait()

        def gather(b):
            pltpu.async_copy(x_hbm.at[srcv[b]], rows[b], gsem[b])

        def gather_wait(b):
            pltpu.make_async_copy(x_hbm.at[srcv[b]], rows[b], gsem[b]).wait()

        def scatter_add(b):
            pltpu.sync_copy(rows[b], agg_sh.at[dstv[b]], add=True)

        idx_load(0, 0)
        idx_load(1, 1)
        idx_wait(0)
        gather(0)

        def pair_body(i, carry):
            c0 = 2 * i
            gather_wait(0)
            idx_wait(1)
            gather(1)
            scatter_add(0)
            deg_update(0)
            idx_load(jnp.minimum(c0 + 2, N_CHUNKS - 1), 0)
            gather_wait(1)
            idx_wait(0)
            gather(0)
            scatter_add(1)
            deg_update(1)
            idx_load(jnp.minimum(c0 + 3, N_CHUNKS - 1), 1)
            return carry

        lax.fori_loop(0, N_CHUNKS // 2, pair_body, 0)
        gather_wait(0)
        idx_wait(1)

        plsc.subcore_barrier()
        pltpu.sync_copy(
            agg_sh.at[pl.ds(sid * R8, R8)],
            part_hbm.at[cid, pl.ds(sid * R8, R8)])

        @pl.when(sid == 1)
        def _():
            pltpu.sync_copy(
                agg_sh.at[pl.ds(NS * R8, TAIL)],
                part_hbm.at[cid, pl.ds(NS * R8, TAIL)])

        pltpu.sync_copy(deg_v.at[pl.ds(0, N_NODES)], degp_hbm.at[wid, 0])

    return k(x, src, dst, zrows)


def _tc_linear(x, part, degp, w1t, w2t, b2):
    G = 1000

    def body(x_ref, p_ref, degp_ref, w1_ref, w2_ref, b_ref, o_ref):
        deg = jnp.sum(degp_ref[...], axis=1)
        inv = 1.0 / jnp.maximum(deg, 1.0)
        h = (p_ref[0] + p_ref[1]) * inv[:, None]
        o_ref[...] = (
            jnp.dot(x_ref[...], w1_ref[...], preferred_element_type=jnp.float32)
            + jnp.dot(h, w2_ref[...], preferred_element_type=jnp.float32)
            + b_ref[...])

    return pl.pallas_call(
        body,
        grid=(N_NODES // G,),
        in_specs=[
            pl.BlockSpec((G, D), lambda i: (i, 0)),
            pl.BlockSpec((NC, G, D), lambda i: (0, i, 0)),
            pl.BlockSpec((G, NW), lambda i: (i, 0)),
            pl.BlockSpec((D, D), lambda i: (0, 0)),
            pl.BlockSpec((D, D), lambda i: (0, 0)),
            pl.BlockSpec((1, D), lambda i: (0, 0)),
        ],
        out_specs=pl.BlockSpec((G, D), lambda i: (i, 0)),
        out_shape=jax.ShapeDtypeStruct((N_NODES, D), jnp.float32),
    )(x, part, degp, w1t, w2t, b2)


def kernel(x, edge_index, W, b):
    npad = E_PAD - N_EDGES
    src = jnp.concatenate(
        [edge_index[0].astype(jnp.int32), jnp.zeros((npad,), jnp.int32)])
    dst = jnp.concatenate(
        [edge_index[1].astype(jnp.int32),
         jnp.full((npad,), N_NODES, jnp.int32)])
    zrows = jnp.zeros((R8, D), jnp.float32)
    part, degp = _sc_aggregate(x, src, dst, zrows)
    degp = degp.reshape(NW, N_NODES).T
    w1t = W[:, :D].T
    w2t = W[:, D:].T
    return _tc_linear(x, part, degp, w1t, w2t, b[None, :])


# speedup vs baseline: 1.0415x; 1.0415x over previous
import functools

import jax
import jax.numpy as jnp
from jax import lax
from jax.experimental import pallas as pl
from jax.experimental.pallas import tpu as pltpu
from jax.experimental.pallas import tpu_sc as plsc

N_NODES = 10000
N_EDGES = 320000
D = 128
NC, NS, L = 2, 16, 16
NW = NC * NS
CHUNK = 128
N_CHUNKS = 80
E_PER_W = N_CHUNKS * CHUNK
E_PAD = NW * E_PER_W
N_ACC = N_NODES + 16
R8 = (N_NODES // NS) // 8 * 8
TAIL = N_NODES - R8 * NS
ZTAIL = N_ACC - R8 * NS


def _sc_aggregate(x, src, dst, zrows):
    mesh = plsc.VectorSubcoreMesh(core_axis_name="c", subcore_axis_name="s")

    @functools.partial(
        pl.kernel,
        out_type=[
            jax.ShapeDtypeStruct((NC, N_NODES, D), jnp.float32),
            jax.ShapeDtypeStruct((NW, 1, N_NODES), jnp.float32),
        ],
        mesh=mesh,
        compiler_params=pltpu.CompilerParams(needs_layout_passes=False),
        scratch_types=[
            [pltpu.VMEM((CHUNK,), jnp.int32)] * 2,
            [pltpu.VMEM((CHUNK,), jnp.int32)] * 2,
            [pltpu.VMEM((CHUNK, D), jnp.float32)] * 2,
            pltpu.VMEM((N_ACC,), jnp.float32),
            [pltpu.SemaphoreType.DMA] * 2,
            [pltpu.SemaphoreType.DMA] * 2,
            pltpu.VMEM_SHARED((N_ACC, D), jnp.float32),
        ],
    )
    def k(x_hbm, src_hbm, dst_hbm, z_hbm, part_hbm, degp_hbm,
          srcv, dstv, rows, deg_v, isem, gsem, agg_sh):
        cid = lax.axis_index("c")
        sid = lax.axis_index("s")
        wid = sid * NC + cid

        pltpu.sync_copy(z_hbm, agg_sh.at[pl.ds(sid * R8, R8)])

        @pl.when(sid == 0)
        def _():
            pltpu.sync_copy(z_hbm.at[pl.ds(0, ZTAIL)],
                            agg_sh.at[pl.ds(NS * R8, ZTAIL)])

        def zdeg(i, carry):
            deg_v[pl.ds(i * L, L)] = jnp.zeros((L,), jnp.float32)
            return carry

        lax.fori_loop(0, N_ACC // L, zdeg, 0)
        plsc.subcore_barrier()

        ones = jnp.ones((L,), jnp.float32)

        def deg_update(b):
            def deg_body(j, c2):
                idx = dstv[b][pl.ds(j * L, L)]
                plsc.addupdate_scatter(deg_v, [idx], ones)
                return c2

            lax.fori_loop(0, CHUNK // L, deg_body, 0)

        def idx_load(c, b):
            base = wid * E_PER_W + c * CHUNK
            pltpu.async_copy(src_hbm.at[pl.ds(base, CHUNK)], srcv[b], isem[b])
            pltpu.async_copy(dst_hbm.at[pl.ds(base, CHUNK)], dstv[b], isem[b])

        def idx_wait(b):
            pltpu.make_async_copy(src_hbm.at[pl.ds(0, CHUNK)], srcv[b], isem[b]).wait()
            pltpu.make_async_copy(dst_hbm.at[pl.ds(0, CHUNK)], dstv[b], isem[b]).wait()

        def gather(b):
            pltpu.async_copy(x_hbm.at[srcv[b]], rows[b], gsem[b])

        def gather_wait(b):
            pltpu.make_async_copy(x_hbm.at[srcv[b]], rows[b], gsem[b]).wait()

        def scatter_add(b):
            pltpu.sync_copy(rows[b], agg_sh.at[dstv[b]], add=True)

        idx_load(0, 0)
        idx_load(1, 1)
        idx_wait(0)
        gather(0)

        def pair_body(i, carry):
            c0 = 2 * i
            gather_wait(0)
            idx_wait(1)
            gather(1)
            scatter_add(0)
            deg_update(0)
            idx_load(jnp.minimum(c0 + 2, N_CHUNKS - 1), 0)
            gather_wait(1)
            idx_wait(0)
            gather(0)
            scatter_add(1)
            deg_update(1)
            idx_load(jnp.minimum(c0 + 3, N_CHUNKS - 1), 1)
            return carry

        lax.fori_loop(0, N_CHUNKS // 2, pair_body, 0)
        gather_wait(0)
        idx_wait(1)

        plsc.subcore_barrier()
        pltpu.sync_copy(
            agg_sh.at[pl.ds(sid * R8, R8)],
            part_hbm.at[cid, pl.ds(sid * R8, R8)])

        @pl.when(sid == 1)
        def _():
            pltpu.sync_copy(
                agg_sh.at[pl.ds(NS * R8, TAIL)],
                part_hbm.at[cid, pl.ds(NS * R8, TAIL)])

        pltpu.sync_copy(deg_v.at[pl.ds(0, N_NODES)], degp_hbm.at[wid, 0])

    return k(x, src, dst, zrows)


def _tc_linear(x, part, degp, w1t, w2t, b2):
    G = 1000

    def body(x_ref, p_ref, degp_ref, w1_ref, w2_ref, b_ref, o_ref):
        deg = jnp.sum(degp_ref[...], axis=1)
        inv = 1.0 / jnp.maximum(deg, 1.0)
        h = (p_ref[0] + p_ref[1]) * inv[:, None]
        o_ref[...] = (
            jnp.dot(x_ref[...], w1_ref[...], preferred_element_type=jnp.float32)
            + jnp.dot(h, w2_ref[...], preferred_element_type=jnp.float32)
            + b_ref[...])

    return pl.pallas_call(
        body,
        grid=(N_NODES // G,),
        in_specs=[
            pl.BlockSpec((G, D), lambda i: (i, 0)),
            pl.BlockSpec((NC, G, D), lambda i: (0, i, 0)),
            pl.BlockSpec((G, NW), lambda i: (i, 0)),
            pl.BlockSpec((D, D), lambda i: (0, 0)),
            pl.BlockSpec((D, D), lambda i: (0, 0)),
            pl.BlockSpec((1, D), lambda i: (0, 0)),
        ],
        out_specs=pl.BlockSpec((G, D), lambda i: (i, 0)),
        out_shape=jax.ShapeDtypeStruct((N_NODES, D), jnp.float32),
    )(x, part, degp, w1t, w2t, b2)


def kernel(x, edge_index, W, b):
    npad = E_PAD - N_EDGES
    src = jnp.concatenate(
        [edge_index[0].astype(jnp.int32), jnp.zeros((npad,), jnp.int32)])
    dst = jnp.concatenate(
        [edge_index[1].astype(jnp.int32),
         N_NODES + jnp.arange(npad, dtype=jnp.int32) % 16])
    zrows = jnp.zeros((R8, D), jnp.float32)
    part, degp = _sc_aggregate(x, src, dst, zrows)
    degp = degp.reshape(NW, N_NODES).T
    w1t = W[:, :D].T
    w2t = W[:, D:].T
    return _tc_linear(x, part, degp, w1t, w2t, b[None, :])


# speedup vs baseline: 3.1764x; 3.0498x over previous
import functools

import jax
import jax.numpy as jnp
from jax import lax
from jax.experimental import pallas as pl
from jax.experimental.pallas import tpu as pltpu
from jax.experimental.pallas import tpu_sc as plsc

N_NODES = 10000
N_EDGES = 320000
D = 128
NC, NS, L = 2, 16, 16
NW = NC * NS
CHUNK = 128
N_CHUNKS = 80
E_PER_W = N_CHUNKS * CHUNK
E_PAD = NW * E_PER_W
N_ACC = N_NODES + 16
R8 = (N_NODES // NS) // 8 * 8
TAIL = N_NODES - R8 * NS
ZTAIL = N_ACC - R8 * NS


def _sc_aggregate(x, src, dst, zrows):
    mesh = plsc.VectorSubcoreMesh(core_axis_name="c", subcore_axis_name="s")

    @functools.partial(
        pl.kernel,
        out_type=[
            jax.ShapeDtypeStruct((NC, N_NODES, D), jnp.float32),
            jax.ShapeDtypeStruct((NW, 1, N_NODES), jnp.float32),
        ],
        mesh=mesh,
        compiler_params=pltpu.CompilerParams(needs_layout_passes=False),
        scratch_types=[
            [pltpu.VMEM((CHUNK,), jnp.int32)] * 2,
            [pltpu.VMEM((CHUNK,), jnp.int32)] * 2,
            [pltpu.VMEM((CHUNK, D), jnp.float32)] * 2,
            pltpu.VMEM((N_ACC,), jnp.float32),
            [pltpu.SemaphoreType.DMA] * 2,
            [pltpu.SemaphoreType.DMA] * 2,
            pltpu.VMEM_SHARED((N_ACC, D), jnp.float32),
        ],
    )
    def k(x_hbm, src_hbm, dst_hbm, z_hbm, part_hbm, degp_hbm,
          srcv, dstv, rows, deg_v, isem, gsem, agg_sh):
        cid = lax.axis_index("c")
        sid = lax.axis_index("s")
        wid = sid * NC + cid

        pltpu.sync_copy(z_hbm, agg_sh.at[pl.ds(sid * R8, R8)])

        @pl.when(sid == 0)
        def _():
            pltpu.sync_copy(z_hbm.at[pl.ds(0, ZTAIL)],
                            agg_sh.at[pl.ds(NS * R8, ZTAIL)])

        def zdeg(i, carry):
            deg_v[pl.ds(i * L, L)] = jnp.zeros((L,), jnp.float32)
            return carry

        lax.fori_loop(0, N_ACC // L, zdeg, 0)
        plsc.subcore_barrier()

        ones = jnp.ones((L,), jnp.float32)

        def deg_update(b):
            def deg_body(j, c2):
                idx = dstv[b][pl.ds(j * L, L)]
                plsc.addupdate_scatter(deg_v, [idx], ones)
                return c2

            lax.fori_loop(0, CHUNK // L, deg_body, 0)

        def idx_load(c, b):
            base = wid * E_PER_W + c * CHUNK
            pltpu.async_copy(src_hbm.at[pl.ds(base, CHUNK)], srcv[b], isem[b])
            pltpu.async_copy(dst_hbm.at[pl.ds(base, CHUNK)], dstv[b], isem[b])

        def idx_wait(b):
            pltpu.make_async_copy(src_hbm.at[pl.ds(0, CHUNK)], srcv[b], isem[b]).wait()
            pltpu.make_async_copy(dst_hbm.at[pl.ds(0, CHUNK)], dstv[b], isem[b]).wait()

        def gather(b):
            pltpu.async_copy(x_hbm.at[srcv[b]], rows[b], gsem[b])

        def gather_wait(b):
            pltpu.make_async_copy(x_hbm.at[srcv[b]], rows[b], gsem[b]).wait()

        def scatter_add(b):
            pltpu.sync_copy(rows[b], agg_sh.at[dstv[b]], add=True)

        idx_load(0, 0)
        idx_load(1, 1)
        idx_wait(0)
        gather(0)

        def pair_body(i, carry):
            c0 = 2 * i
            gather_wait(0)
            idx_wait(1)
            gather(1)
            scatter_add(0)
            deg_update(0)
            idx_load(jnp.minimum(c0 + 2, N_CHUNKS - 1), 0)
            gather_wait(1)
            idx_wait(0)
            gather(0)
            scatter_add(1)
            deg_update(1)
            idx_load(jnp.minimum(c0 + 3, N_CHUNKS - 1), 1)
            return carry

        lax.fori_loop(0, N_CHUNKS // 2, pair_body, 0)
        gather_wait(0)
        idx_wait(1)

        plsc.subcore_barrier()
        pltpu.sync_copy(
            agg_sh.at[pl.ds(sid * R8, R8)],
            part_hbm.at[cid, pl.ds(sid * R8, R8)])

        @pl.when(sid == 1)
        def _():
            pltpu.sync_copy(
                agg_sh.at[pl.ds(NS * R8, TAIL)],
                part_hbm.at[cid, pl.ds(NS * R8, TAIL)])

        pltpu.sync_copy(deg_v.at[pl.ds(0, N_NODES)], degp_hbm.at[wid, 0])

    return k(x, src, dst, zrows)


def _tc_linear(x, part, degp, w1t, w2t, b2):
    G = 1000

    def body(x_ref, p_ref, degp_ref, w1_ref, w2_ref, b_ref, o_ref):
        deg = jnp.sum(degp_ref[...], axis=1)
        inv = 1.0 / jnp.maximum(deg, 1.0)
        h = (p_ref[0] + p_ref[1]) * inv[:, None]
        o_ref[...] = (
            jnp.dot(x_ref[...], w1_ref[...], preferred_element_type=jnp.float32)
            + jnp.dot(h, w2_ref[...], preferred_element_type=jnp.float32)
            + b_ref[...])

    return pl.pallas_call(
        body,
        grid=(N_NODES // G,),
        in_specs=[
            pl.BlockSpec((G, D), lambda i: (i, 0)),
            pl.BlockSpec((NC, G, D), lambda i: (0, i, 0)),
            pl.BlockSpec((G, NW), lambda i: (i, 0)),
            pl.BlockSpec((D, D), lambda i: (0, 0)),
            pl.BlockSpec((D, D), lambda i: (0, 0)),
            pl.BlockSpec((1, D), lambda i: (0, 0)),
        ],
        out_specs=pl.BlockSpec((G, D), lambda i: (i, 0)),
        out_shape=jax.ShapeDtypeStruct((N_NODES, D), jnp.float32),
    )(x, part, degp, w1t, w2t, b2)


def kernel(x, edge_index, W, b):
    padw = E_PER_W - N_EDGES // NW
    ew = N_EDGES // NW
    srcw = edge_index[0].astype(jnp.int32).reshape(NW, ew)
    dstw = edge_index[1].astype(jnp.int32).reshape(NW, ew)
    padsrc = jnp.broadcast_to(
        (jnp.arange(padw, dtype=jnp.int32) * 37) % N_NODES, (NW, padw))
    paddst = jnp.broadcast_to(
        N_NODES + jnp.arange(padw, dtype=jnp.int32) % 16, (NW, padw))
    src = jnp.concatenate([srcw, padsrc], axis=1).reshape(-1)
    dst = jnp.concatenate([dstw, paddst], axis=1).reshape(-1)
    zrows = jnp.zeros((R8, D), jnp.float32)
    part, degp = _sc_aggregate(x, src, dst, zrows)
    degp = degp.reshape(NW, N_NODES).T
    w1t = W[:, :D].T
    w2t = W[:, D:].T
    return _tc_linear(x, part, degp, w1t, w2t, b[None, :])


# speedup vs baseline: 3.4009x; 1.0707x over previous
import functools

import jax
import jax.numpy as jnp
from jax import lax
from jax.experimental import pallas as pl
from jax.experimental.pallas import tpu as pltpu
from jax.experimental.pallas import tpu_sc as plsc

N_NODES = 10000
N_EDGES = 320000
D = 128
NC, NS, L = 2, 16, 16
NW = NC * NS
CHUNK = 96
N_CHUNKS = 105
E_PER_W = N_CHUNKS * CHUNK
E_PAD = NW * E_PER_W
N_ACC = N_NODES + 16
R8 = (N_NODES // NS) // 8 * 8
TAIL = N_NODES - R8 * NS
ZTAIL = N_ACC - R8 * NS


def _sc_aggregate(x, src, dst, zrows):
    mesh = plsc.VectorSubcoreMesh(core_axis_name="c", subcore_axis_name="s")

    @functools.partial(
        pl.kernel,
        out_type=[
            jax.ShapeDtypeStruct((NC, N_NODES, D), jnp.float32),
            jax.ShapeDtypeStruct((NW, 1, N_NODES), jnp.float32),
        ],
        mesh=mesh,
        compiler_params=pltpu.CompilerParams(needs_layout_passes=False),
        scratch_types=[
            [pltpu.VMEM((CHUNK,), jnp.int32)] * 3,
            [pltpu.VMEM((CHUNK,), jnp.int32)] * 3,
            [pltpu.VMEM((CHUNK, D), jnp.float32)] * 3,
            pltpu.VMEM((N_ACC,), jnp.float32),
            [pltpu.SemaphoreType.DMA] * 3,
            [pltpu.SemaphoreType.DMA] * 3,
            pltpu.VMEM_SHARED((N_ACC, D), jnp.float32),
        ],
    )
    def k(x_hbm, src_hbm, dst_hbm, z_hbm, part_hbm, degp_hbm,
          srcv, dstv, rows, deg_v, isem, gsem, agg_sh):
        cid = lax.axis_index("c")
        sid = lax.axis_index("s")
        wid = sid * NC + cid

        pltpu.sync_copy(z_hbm, agg_sh.at[pl.ds(sid * R8, R8)])

        @pl.when(sid == 0)
        def _():
            pltpu.sync_copy(z_hbm.at[pl.ds(0, ZTAIL)],
                            agg_sh.at[pl.ds(NS * R8, ZTAIL)])

        def zdeg(i, carry):
            deg_v[pl.ds(i * L, L)] = jnp.zeros((L,), jnp.float32)
            return carry

        lax.fori_loop(0, N_ACC // L, zdeg, 0)
        plsc.subcore_barrier()

        ones = jnp.ones((L,), jnp.float32)

        def deg_update(b):
            def deg_body(j, c2):
                idx = dstv[b][pl.ds(j * L, L)]
                plsc.addupdate_scatter(deg_v, [idx], ones)
                return c2

            lax.fori_loop(0, CHUNK // L, deg_body, 0)

        def idx_load(c, b):
            base = wid * E_PER_W + c * CHUNK
            pltpu.async_copy(src_hbm.at[pl.ds(base, CHUNK)], srcv[b], isem[b])
            pltpu.async_copy(dst_hbm.at[pl.ds(base, CHUNK)], dstv[b], isem[b])

        def idx_wait(b):
            pltpu.make_async_copy(src_hbm.at[pl.ds(0, CHUNK)], srcv[b], isem[b]).wait()
            pltpu.make_async_copy(dst_hbm.at[pl.ds(0, CHUNK)], dstv[b], isem[b]).wait()

        def gather(b):
            pltpu.async_copy(x_hbm.at[srcv[b]], rows[b], gsem[b])

        def gather_wait(b):
            pltpu.make_async_copy(x_hbm.at[srcv[b]], rows[b], gsem[b]).wait()

        def scatter_add(b):
            pltpu.sync_copy(rows[b], agg_sh.at[dstv[b]], add=True)

        idx_load(0, 0)
        idx_load(1, 1)
        idx_load(2, 2)
        idx_wait(0)
        gather(0)
        idx_wait(1)
        gather(1)

        def ring_body(i, carry):
            c0 = 3 * i
            for k in range(3):
                b = k
                c = c0 + k
                gather_wait(b)
                scatter_add(b)
                deg_update(b)
                idx_load(jnp.minimum(c + 3, N_CHUNKS - 1), b)
                idx_wait((b + 2) % 3)
                gather((b + 2) % 3)
            return carry

        lax.fori_loop(0, N_CHUNKS // 3, ring_body, 0)
        gather_wait(0)
        gather_wait(1)
        idx_wait(2)

        plsc.subcore_barrier()
        pltpu.sync_copy(
            agg_sh.at[pl.ds(sid * R8, R8)],
            part_hbm.at[cid, pl.ds(sid * R8, R8)])

        @pl.when(sid == 1)
        def _():
            pltpu.sync_copy(
                agg_sh.at[pl.ds(NS * R8, TAIL)],
                part_hbm.at[cid, pl.ds(NS * R8, TAIL)])

        pltpu.sync_copy(deg_v.at[pl.ds(0, N_NODES)], degp_hbm.at[wid, 0])

    return k(x, src, dst, zrows)


def _tc_linear(x, part, degp, w1t, w2t, b2):
    G = 1000

    def body(x_ref, p_ref, degp_ref, w1_ref, w2_ref, b_ref, o_ref):
        deg = jnp.sum(degp_ref[...], axis=1)
        inv = 1.0 / jnp.maximum(deg, 1.0)
        h = (p_ref[0] + p_ref[1]) * inv[:, None]
        o_ref[...] = (
            jnp.dot(x_ref[...], w1_ref[...], preferred_element_type=jnp.float32)
            + jnp.dot(h, w2_ref[...], preferred_element_type=jnp.float32)
            + b_ref[...])

    return pl.pallas_call(
        body,
        grid=(N_NODES // G,),
        in_specs=[
            pl.BlockSpec((G, D), lambda i: (i, 0)),
            pl.BlockSpec((NC, G, D), lambda i: (0, i, 0)),
            pl.BlockSpec((G, NW), lambda i: (i, 0)),
            pl.BlockSpec((D, D), lambda i: (0, 0)),
            pl.BlockSpec((D, D), lambda i: (0, 0)),
            pl.BlockSpec((1, D), lambda i: (0, 0)),
        ],
        out_specs=pl.BlockSpec((G, D), lambda i: (i, 0)),
        out_shape=jax.ShapeDtypeStruct((N_NODES, D), jnp.float32),
    )(x, part, degp, w1t, w2t, b2)


def kernel(x, edge_index, W, b):
    padw = E_PER_W - N_EDGES // NW
    ew = N_EDGES // NW
    srcw = edge_index[0].astype(jnp.int32).reshape(NW, ew)
    dstw = edge_index[1].astype(jnp.int32).reshape(NW, ew)
    padsrc = jnp.broadcast_to(
        (jnp.arange(padw, dtype=jnp.int32) * 37) % N_NODES, (NW, padw))
    paddst = jnp.broadcast_to(
        N_NODES + jnp.arange(padw, dtype=jnp.int32) % 16, (NW, padw))
    src = jnp.concatenate([srcw, padsrc], axis=1).reshape(-1)
    dst = jnp.concatenate([dstw, paddst], axis=1).reshape(-1)
    zrows = jnp.zeros((R8, D), jnp.float32)
    part, degp = _sc_aggregate(x, src, dst, zrows)
    degp = degp.reshape(NW, N_NODES).T
    w1t = W[:, :D].T
    w2t = W[:, D:].T
    return _tc_linear(x, part, degp, w1t, w2t, b[None, :])


# speedup vs baseline: 3.6061x; 1.0604x over previous
import functools

import jax
import jax.numpy as jnp
from jax import lax
from jax.experimental import pallas as pl
from jax.experimental.pallas import tpu as pltpu
from jax.experimental.pallas import tpu_sc as plsc

N_NODES = 10000
N_EDGES = 320000
D = 128
NC, NS, L = 2, 16, 16
NW = NC * NS
CHUNK = 72
N_CHUNKS = 140
E_PER_W = N_CHUNKS * CHUNK
E_PAD = NW * E_PER_W
N_ACC = N_NODES + 16
R8 = (N_NODES // NS) // 8 * 8
TAIL = N_NODES - R8 * NS
ZTAIL = N_ACC - R8 * NS


def _sc_aggregate(x, src, dst, zrows):
    mesh = plsc.VectorSubcoreMesh(core_axis_name="c", subcore_axis_name="s")

    @functools.partial(
        pl.kernel,
        out_type=[
            jax.ShapeDtypeStruct((NC, N_NODES, D), jnp.float32),
            jax.ShapeDtypeStruct((NW, 1, N_NODES), jnp.float32),
        ],
        mesh=mesh,
        compiler_params=pltpu.CompilerParams(needs_layout_passes=False),
        scratch_types=[
            [pltpu.VMEM((CHUNK,), jnp.int32)] * 4,
            [pltpu.VMEM((CHUNK,), jnp.int32)] * 4,
            [pltpu.VMEM((CHUNK, D), jnp.float32)] * 4,
            pltpu.VMEM((N_ACC,), jnp.float32),
            [pltpu.SemaphoreType.DMA] * 4,
            [pltpu.SemaphoreType.DMA] * 4,
            pltpu.VMEM_SHARED((N_ACC, D), jnp.float32),
        ],
    )
    def k(x_hbm, src_hbm, dst_hbm, z_hbm, part_hbm, degp_hbm,
          srcv, dstv, rows, deg_v, isem, gsem, agg_sh):
        cid = lax.axis_index("c")
        sid = lax.axis_index("s")
        wid = sid * NC + cid

        pltpu.sync_copy(z_hbm, agg_sh.at[pl.ds(sid * R8, R8)])

        @pl.when(sid == 0)
        def _():
            pltpu.sync_copy(z_hbm.at[pl.ds(0, ZTAIL)],
                            agg_sh.at[pl.ds(NS * R8, ZTAIL)])

        def zdeg(i, carry):
            deg_v[pl.ds(i * L, L)] = jnp.zeros((L,), jnp.float32)
            return carry

        lax.fori_loop(0, N_ACC // L, zdeg, 0)
        plsc.subcore_barrier()

        ones = jnp.ones((L,), jnp.float32)

        def deg_update(b):
            def deg_body(j, c2):
                idx = dstv[b][pl.ds(j * L, L)]
                plsc.addupdate_scatter(deg_v, [idx], ones)
                return c2

            lax.fori_loop(0, CHUNK // L, deg_body, 0)

        def idx_load(c, b):
            base = wid * E_PER_W + c * CHUNK
            pltpu.async_copy(src_hbm.at[pl.ds(base, CHUNK)], srcv[b], isem[b])
            pltpu.async_copy(dst_hbm.at[pl.ds(base, CHUNK)], dstv[b], isem[b])

        def idx_wait(b):
            pltpu.make_async_copy(src_hbm.at[pl.ds(0, CHUNK)], srcv[b], isem[b]).wait()
            pltpu.make_async_copy(dst_hbm.at[pl.ds(0, CHUNK)], dstv[b], isem[b]).wait()

        def gather(b):
            pltpu.async_copy(x_hbm.at[srcv[b]], rows[b], gsem[b])

        def gather_wait(b):
            pltpu.make_async_copy(x_hbm.at[srcv[b]], rows[b], gsem[b]).wait()

        def scatter_add(b):
            pltpu.sync_copy(rows[b], agg_sh.at[dstv[b]], add=True)

        NB = 4
        for b0 in range(NB):
            idx_load(b0, b0)
        for b0 in range(NB - 1):
            idx_wait(b0)
            gather(b0)

        def ring_body(i, carry):
            c0 = NB * i
            for k in range(NB):
                b = k
                c = c0 + k
                gather_wait(b)
                scatter_add(b)
                deg_update(b)
                idx_load(jnp.minimum(c + NB, N_CHUNKS - 1), b)
                idx_wait((b + NB - 1) % NB)
                gather((b + NB - 1) % NB)
            return carry

        lax.fori_loop(0, N_CHUNKS // NB, ring_body, 0)
        for b0 in range(NB - 1):
            gather_wait(b0)
        idx_wait(NB - 1)

        plsc.subcore_barrier()
        pltpu.sync_copy(
            agg_sh.at[pl.ds(sid * R8, R8)],
            part_hbm.at[cid, pl.ds(sid * R8, R8)])

        @pl.when(sid == 1)
        def _():
            pltpu.sync_copy(
                agg_sh.at[pl.ds(NS * R8, TAIL)],
                part_hbm.at[cid, pl.ds(NS * R8, TAIL)])

        pltpu.sync_copy(deg_v.at[pl.ds(0, N_NODES)], degp_hbm.at[wid, 0])

    return k(x, src, dst, zrows)


def _tc_linear(x, part, degp, w1t, w2t, b2):
    G = 1000

    def body(x_ref, p_ref, degp_ref, w1_ref, w2_ref, b_ref, o_ref):
        deg = jnp.sum(degp_ref[...], axis=1)
        inv = 1.0 / jnp.maximum(deg, 1.0)
        h = (p_ref[0] + p_ref[1]) * inv[:, None]
        o_ref[...] = (
            jnp.dot(x_ref[...], w1_ref[...], preferred_element_type=jnp.float32)
            + jnp.dot(h, w2_ref[...], preferred_element_type=jnp.float32)
            + b_ref[...])

    return pl.pallas_call(
        body,
        grid=(N_NODES // G,),
        in_specs=[
            pl.BlockSpec((G, D), lambda i: (i, 0)),
            pl.BlockSpec((NC, G, D), lambda i: (0, i, 0)),
            pl.BlockSpec((G, NW), lambda i: (i, 0)),
            pl.BlockSpec((D, D), lambda i: (0, 0)),
            pl.BlockSpec((D, D), lambda i: (0, 0)),
            pl.BlockSpec((1, D), lambda i: (0, 0)),
        ],
        out_specs=pl.BlockSpec((G, D), lambda i: (i, 0)),
        out_shape=jax.ShapeDtypeStruct((N_NODES, D), jnp.float32),
    )(x, part, degp, w1t, w2t, b2)


def kernel(x, edge_index, W, b):
    padw = E_PER_W - N_EDGES // NW
    ew = N_EDGES // NW
    srcw = edge_index[0].astype(jnp.int32).reshape(NW, ew)
    dstw = edge_index[1].astype(jnp.int32).reshape(NW, ew)
    padsrc = jnp.broadcast_to(
        (jnp.arange(padw, dtype=jnp.int32) * 37) % N_NODES, (NW, padw))
    paddst = jnp.broadcast_to(
        N_NODES + jnp.arange(padw, dtype=jnp.int32) % 16, (NW, padw))
    src = jnp.concatenate([srcw, padsrc], axis=1).reshape(-1)
    dst = jnp.concatenate([dstw, paddst], axis=1).reshape(-1)
    zrows = jnp.zeros((R8, D), jnp.float32)
    part, degp = _sc_aggregate(x, src, dst, zrows)
    degp = degp.reshape(NW, N_NODES).T
    w1t = W[:, :D].T
    w2t = W[:, D:].T
    return _tc_linear(x, part, degp, w1t, w2t, b[None, :])
